# speedup vs baseline: 5.3280x; 5.3280x over previous
import jax
import jax.numpy as jnp
from jax import lax
from jax.experimental import pallas as pl
from jax.experimental.pallas import tpu as pltpu

B = 16
NA = 3
NC = 80
G = 80
NT = 128
STRIDE = 8.0
AW = (10.0, 16.0, 33.0)
AH = (13.0, 30.0, 23.0)
ROWS = B * NA * G


def _softplus(x):
    return jnp.maximum(x, 0.0) + jnp.log1p(jnp.exp(-jnp.abs(x)))


def _assign(t, axis):
    if axis == 0:
        def col(k):
            return t[:, k:k + 1]
    else:
        def col(k):
            return t[k:k + 1, :]
    imgf, clsf = col(0), col(1)
    tbx, tby, tbw, tbh = col(2), col(3), col(4), col(5)
    b_i = imgf.astype(jnp.int32)
    cls_i = clsf.astype(jnp.int32)
    cx = (tbx * float(G)).astype(jnp.int32)
    cy = (tby * float(G)).astype(jnp.int32)
    tw = tbw * float(G)
    th = tbh * float(G)
    best = jnp.zeros_like(b_i)
    m = None
    for k in range(NA):
        aw = AW[k] / STRIDE
        ah = AH[k] / STRIDE
        inter = jnp.minimum(tw, aw) * jnp.minimum(th, ah)
        iou = inter / (tw * th + aw * ah - inter + 1e-6)
        if m is None:
            m = iou
        else:
            best = jnp.where(iou > m, k, best)
            m = jnp.maximum(m, iou)
    valid = ((b_i >= 0) & (b_i < B)
             & (cx >= 0) & (cx < G) & (cy >= 0) & (cy < G))
    return dict(b=b_i, cls=cls_i, cx=cx, cy=cy, best=best, valid=valid,
                tbx=tbx, tby=tby, tbw=tbw, tbh=tbh)


def _meta_body(t_ref, m_ref):
    q = _assign(t_ref[...], axis=0)
    bc = jnp.clip(q["b"], 0, B - 1)
    cyc = jnp.clip(q["cy"], 0, G - 1)
    cxc = jnp.clip(q["cx"], 0, G - 1)
    row = (bc * NA + q["best"]) * G + cyc
    v = q["valid"].astype(jnp.int32)
    m_ref[:, 0:1] = bc
    m_ref[:, 1:2] = q["best"]
    m_ref[:, 2:3] = cyc
    m_ref[:, 3:4] = cxc
    m_ref[:, 4:5] = v
    m_ref[:, 5:6] = row
    m_ref[:, 6:8] = jnp.zeros((NT, 2), jnp.int32)


def _main_body(pred_ref, tc_ref, tr_ref, meta_ref, out_ref,
               obj_ref, gbuf_ref, sem_obj, sem_g):
    obj_copies = []
    for b in range(B):
        for a in range(NA):
            c = pltpu.make_async_copy(
                pred_ref.at[b, 4 + 85 * a],
                obj_ref.at[pl.ds((b * NA + a) * G, G)], sem_obj)
            c.start()
            obj_copies.append(c)
    g_copies = []
    for t in range(NT):
        tb = meta_ref[t, 0]
        ta = meta_ref[t, 1]
        tcy = meta_ref[t, 2]
        c = pltpu.make_async_copy(
            pred_ref.at[tb, pl.ds(ta * 85, 85), tcy, :],
            gbuf_ref.at[t], sem_g)
        c.start()
        g_copies.append(c)

    qc = _assign(tc_ref[...], axis=0)
    qr = _assign(tr_ref[...], axis=1)
    tcol = lax.broadcasted_iota(jnp.int32, (NT, NT), 0)
    trow = lax.broadcasted_iota(jnp.int32, (NT, NT), 1)
    tid_c = lax.broadcasted_iota(jnp.int32, (NT, 1), 0)
    tid_r = lax.broadcasted_iota(jnp.int32, (1, NT), 1)

    def keys(q, tid):
        key = jnp.where(q["valid"],
                        (q["b"] * NA + q["best"]) * (G * G) + q["cy"] * G + q["cx"],
                        -1 - tid)
        cvalid = q["valid"] & (q["cls"] >= 0) & (q["cls"] < NC)
        pkey = jnp.where(cvalid, key * NC + q["cls"], -1 - tid)
        return key, pkey, cvalid

    key_c, pkey_c, cvalid_c = keys(qc, tid_c)
    key_r, pkey_r, cvalid_r = keys(qr, tid_r)
    dup = (key_c == key_r) & (trow > tcol) & qr["valid"]
    winner = qc["valid"] & jnp.logical_not(jnp.any(dup, axis=1, keepdims=True))
    dupp = (pkey_c == pkey_r) & (trow > tcol) & cvalid_r
    upair = cvalid_c & jnp.logical_not(jnp.any(dupp, axis=1, keepdims=True))
    wf = winner.astype(jnp.float32)
    uf = upair.astype(jnp.float32)

    for c in g_copies:
        c.wait()
    cx3 = jnp.clip(qc["cx"], 0, G - 1).reshape(NT, 1, 1)
    lane3 = lax.broadcasted_iota(jnp.int32, (NT, 85, G), 2)
    g = jnp.sum(jnp.where(lane3 == cx3, gbuf_ref[...], 0.0), axis=2)
    px, py = g[:, 0:1], g[:, 1:2]
    pw, ph, pobj = g[:, 2:3], g[:, 3:4], g[:, 4:5]
    pcls = g[:, 5:85]

    obj_sub = jnp.sum(wf * pobj)
    cls_sp = jnp.sum(wf * _softplus(pcls))
    onehot = lax.broadcasted_iota(jnp.int32, (NT, NC), 1) == qc["cls"]
    cls_sub = jnp.sum(jnp.where(onehot, pcls, 0.0) * uf)

    awb = jnp.where(qc["best"] == 0, AW[0], jnp.where(qc["best"] == 1, AW[1], AW[2]))
    ahb = jnp.where(qc["best"] == 0, AH[0], jnp.where(qc["best"] == 1, AH[1], AH[2]))
    tx = qc["tbx"] / STRIDE - qc["cx"].astype(jnp.float32)
    ty = qc["tby"] / STRIDE - qc["cy"].astype(jnp.float32)
    tw_t = jnp.log(qc["tbw"] / awb + 1e-6)
    th_t = jnp.log(qc["tbh"] / ahb + 1e-6)
    sx = 1.0 / (1.0 + jnp.exp(-px))
    sy = 1.0 / (1.0 + jnp.exp(-py))
    box = jnp.sum(wf * ((sx - tx) ** 2 + (sy - ty) ** 2
                        + (pw - tw_t) ** 2 + (ph - th_t) ** 2))

    sparse = -obj_sub + 0.5 * (cls_sp - cls_sub) + box

    for c in obj_copies:
        c.wait()
    sp = _softplus(obj_ref[...])
    s_total = sparse + jnp.sum(sp)
    out_ref[...] = s_total + 0.5 * sp

    lane80 = lax.broadcasted_iota(jnp.int32, (1, G), 1)

    def fixup(t, carry):
        r = meta_ref[t, 5]
        cxt = meta_ref[t, 3]
        v = meta_ref[t, 4]
        rowv = out_ref[pl.ds(r, 1), :]
        out_ref[pl.ds(r, 1), :] = jnp.where((lane80 == cxt) & (v > 0),
                                            s_total, rowv)
        return carry

    lax.fori_loop(0, NT, fixup, 0)


@jax.jit
def kernel(predictions, targets):
    meta = pl.pallas_call(
        _meta_body,
        out_shape=jax.ShapeDtypeStruct((NT, 8), jnp.int32),
    )(targets)
    out_flat = pl.pallas_call(
        _main_body,
        out_shape=jax.ShapeDtypeStruct((ROWS, G), jnp.float32),
        in_specs=[
            pl.BlockSpec(memory_space=pltpu.MemorySpace.HBM),
            pl.BlockSpec(memory_space=pltpu.MemorySpace.VMEM),
            pl.BlockSpec(memory_space=pltpu.MemorySpace.VMEM),
            pl.BlockSpec(memory_space=pltpu.MemorySpace.SMEM),
        ],
        scratch_shapes=[
            pltpu.VMEM((ROWS, G), jnp.float32),
            pltpu.VMEM((NT, 85, G), jnp.float32),
            pltpu.SemaphoreType.DMA,
            pltpu.SemaphoreType.DMA,
        ],
    )(predictions, targets, jnp.transpose(targets), meta)
    return out_flat.reshape(B, NA, G, G)
